# baseline (speedup 1.0000x reference)
import functools

import jax
import jax.numpy as jnp
from jax.experimental import pallas as pl
from jax.experimental.pallas import tpu as pltpu

B = 128
V = 100000
K = 100
ROWS_PER_BLOCK = 8
EPS = 1e-8
NEG = -jnp.inf


def _body(s_ref, t_ref, out_ref, scratch):
    pid = pl.program_id(0)

    scratch[...] = t_ref[...]
    s_blk = s_ref[...]

    col = jax.lax.broadcasted_iota(jnp.int32, (ROWS_PER_BLOCK, V), 1)

    def step(_, carry):
        dot, ss, tt = carry
        x = scratch[...]
        m = jnp.max(x, axis=1, keepdims=True)
        eq = x == m
        idx = jnp.min(jnp.where(eq, col, V), axis=1, keepdims=True)
        hit = col == idx
        sval = jnp.sum(jnp.where(hit, s_blk, 0.0), axis=1, keepdims=True)
        scratch[...] = jnp.where(hit, NEG, x)
        return (dot + sval * m, ss + sval * sval, tt + m * m)

    zeros = jnp.zeros((ROWS_PER_BLOCK, 1), jnp.float32)
    dot, ss, tt = jax.lax.fori_loop(0, K, step, (zeros, zeros, zeros))

    nx = jnp.maximum(jnp.sqrt(ss), EPS)
    ny = jnp.maximum(jnp.sqrt(tt), EPS)
    cos = dot / (nx * ny)
    part = (jnp.sum(1.0 - cos) / B).reshape(1, 1)

    @pl.when(pid == 0)
    def _():
        out_ref[...] = jnp.zeros_like(out_ref)

    out_ref[...] += part


def kernel(s_logits, t_logits):
    grid = (B // ROWS_PER_BLOCK,)
    out = pl.pallas_call(
        _body,
        grid=grid,
        in_specs=[
            pl.BlockSpec((ROWS_PER_BLOCK, V), lambda i: (i, 0)),
            pl.BlockSpec((ROWS_PER_BLOCK, V), lambda i: (i, 0)),
        ],
        out_specs=pl.BlockSpec((1, 1), lambda i: (0, 0)),
        out_shape=jax.ShapeDtypeStruct((1, 1), jnp.float32),
        scratch_shapes=[pltpu.VMEM((ROWS_PER_BLOCK, V), jnp.float32)],
    )(s_logits, t_logits)
    return out[0, 0]


# speedup vs baseline: 7.1479x; 7.1479x over previous
import functools

import jax
import jax.numpy as jnp
from jax import lax
from jax.experimental import pallas as pl
from jax.experimental.pallas import tpu as pltpu
from jax.experimental.pallas import tpu_sc as plsc

B = 128
V = 100000
K = 100
EPS = 1e-8

NC = 2
NS = 16
NW = NC * NS
RPW = B // NW
WIN = 25000
NWIN = V // WIN
CAP = 1024
GRP = 64
NGRP = WIN // GRP
GV = 112
L = 16


def _sc_body(s_hbm, t_hbm, out_hbm, win, cval, cidx, pbuf, gval, gidx,
             sdst, res, sem):
    wid = lax.axis_index("s") * NC + lax.axis_index("c")
    lane = lax.iota(jnp.int32, 16)

    def select_thr(n):
        def cp(i, c):
            pbuf[pl.ds(i * 16, 16)] = cval[pl.ds(i * 16, 16)]
            return c

        lax.fori_loop(0, CAP // 16, cp, 0)

        def w_cond(c):
            return jnp.logical_not(c[0])

        def w_body(c):
            _, an, r, thr, m = c
            v0 = pbuf[pl.ds(0, 16)]
            pivot = jnp.sum(jnp.where(lane == 0, v0, 0.0))
            nv = (an + 15) // 16

            def cnt(i, cc):
                cgt, ceq = cc
                x = pbuf[pl.ds(i * 16, 16)]
                valid = (lane + i * 16) < an
                gt = jnp.logical_and(valid, x > pivot)
                eq = jnp.logical_and(valid, x == pivot)
                return (cgt + jnp.sum(gt.astype(jnp.int32)),
                        ceq + jnp.sum(eq.astype(jnp.int32)))

            cgt, ceq = lax.fori_loop(0, nv, cnt,
                                     (jnp.int32(0), jnp.int32(0)))
            done = jnp.logical_and(cgt <= r, cgt + ceq > r)

            def finish(an, r, thr, m):
                return (jnp.bool_(True), an, r, pivot, r - cgt + 1)

            def recurse(an, r, thr, m):
                up = cgt > r

                def cmp(i, kp):
                    x = pbuf[pl.ds(i * 16, 16)]
                    valid = (lane + i * 16) < an
                    gt = jnp.logical_and(valid, x > pivot)
                    lt = jnp.logical_and(valid, x < pivot)
                    keep = jnp.logical_or(
                        jnp.logical_and(gt, up),
                        jnp.logical_and(lt, jnp.logical_not(up)))
                    cum = plsc.cumsum(keep.astype(jnp.int32))
                    pos = kp + cum - 1
                    plsc.store_scatter(pbuf, [pos], x, mask=keep)
                    return kp + jnp.sum(keep.astype(jnp.int32))

                kp = lax.fori_loop(0, nv, cmp, jnp.int32(0))
                r2 = jnp.where(up, r, r - (cgt + ceq))
                return (jnp.bool_(False), kp, r2, thr, m)

            return lax.cond(done, finish, recurse, an, r, thr, m)

        carry = (jnp.bool_(False), n, jnp.int32(K - 1), jnp.float32(0.0),
                 jnp.int32(0))
        _, _, _, thr, m = lax.while_loop(w_cond, w_body, carry)
        return thr, m

    def compact(n, thr, m, dval, didx):
        nv = (n + 15) // 16

        def body(i, cc):
            kp, eqt = cc
            x = cval[pl.ds(i * 16, 16)]
            ix = cidx[pl.ds(i * 16, 16)]
            valid = (lane + i * 16) < n
            gt = jnp.logical_and(valid, x > thr)
            eq = jnp.logical_and(valid, x == thr)
            eqc = plsc.cumsum(eq.astype(jnp.int32)) + eqt
            take = jnp.logical_and(eq, eqc <= m)
            keep = jnp.logical_or(gt, take)
            cum = plsc.cumsum(keep.astype(jnp.int32))
            pos = kp + cum - 1
            plsc.store_scatter(dval, [pos], x, mask=keep)
            plsc.store_scatter(didx, [pos], ix, mask=keep)
            return (kp + jnp.sum(keep.astype(jnp.int32)),
                    eqt + jnp.sum(eq.astype(jnp.int32)))

        lax.fori_loop(0, nv, body, (jnp.int32(0), jnp.int32(0)))

    def row_body(j, _):
        r = wid * RPW + j
        sbase = r * V

        def win_body(w, carry):
            ptr, thr = carry
            pltpu.sync_copy(t_hbm.at[pl.ds(sbase + w * WIN, WIN)], win)

            def grp(g, carry):
                ptr, thr = carry
                off = g * GRP
                xs = [win[pl.ds(off + q * 16, 16)] for q in range(GRP // 16)]
                anyv = xs[0] > thr
                for q in range(1, GRP // 16):
                    anyv = jnp.logical_or(anyv, xs[q] > thr)
                hit = jnp.any(anyv)

                def rebuild(ptr0, thr0):
                    t2, m2 = select_thr(ptr0)
                    compact(ptr0, t2, m2, cval, cidx)
                    return jnp.int32(K), t2

                def keep_pt(ptr0, thr0):
                    return ptr0, thr0

                full = jnp.logical_and(hit, ptr + GRP > CAP)
                ptr, thr = lax.cond(full, rebuild, keep_pt, ptr, thr)

                def do_append(p):
                    pp = p
                    for q in range(GRP // 16):
                        x = xs[q]
                        mm = x > thr
                        cum = plsc.cumsum(mm.astype(jnp.int32))
                        pos = pp + cum - 1
                        plsc.store_scatter(cval, [pos], x, mask=mm)
                        colv = sbase + w * WIN + off + q * 16 + lane
                        plsc.store_scatter(cidx, [pos], colv, mask=mm)
                        pp = pp + jnp.sum(mm.astype(jnp.int32))
                    return pp

                ptr = lax.cond(hit, do_append, lambda p: p, ptr)
                return ptr, thr

            return lax.fori_loop(0, NGRP, grp, (ptr, thr))

        ptr, thr = lax.fori_loop(0, NWIN, win_body,
                                 (jnp.int32(0), jnp.float32(-jnp.inf)))

        for q in range(GV // 16):
            gval[pl.ds(q * 16, 16)] = jnp.zeros((16,), jnp.float32)
            gidx[pl.ds(q * 16, 16)] = jnp.zeros((16,), jnp.int32) + sbase

        t2, m2 = select_thr(ptr)
        compact(ptr, t2, m2, gval, gidx)

        pltpu.async_copy(s_hbm.at[gidx], sdst, sem).wait()

        def acc(q, cc):
            d, ss, tt = cc
            tv = gval[pl.ds(q * 16, 16)]
            sv = sdst[pl.ds(q * 16, 16)]
            valid = (lane + q * 16) < K
            sv = jnp.where(valid, sv, 0.0)
            return d + sv * tv, ss + sv * sv, tt + tv * tv

        z = jnp.zeros((16,), jnp.float32)
        dv, ssv, ttv = lax.fori_loop(0, GV // 16, acc, (z, z, z))
        dot = jnp.sum(dv)
        ssum = jnp.sum(ssv)
        tsum = jnp.sum(ttv)
        res[...] = jnp.where(
            lane == 0, dot,
            jnp.where(lane == 1, ssum, jnp.where(lane == 2, tsum, 0.0)))
        pltpu.sync_copy(res, out_hbm.at[pl.ds(r * L, L)])
        return 0

    lax.fori_loop(0, RPW, row_body, 0)


@functools.partial(
    pl.kernel,
    out_type=jax.ShapeDtypeStruct((B * L,), jnp.float32),
    mesh=plsc.VectorSubcoreMesh(core_axis_name="c", subcore_axis_name="s"),
    compiler_params=pltpu.CompilerParams(needs_layout_passes=False),
    scratch_types=[
        pltpu.VMEM((WIN,), jnp.float32),
        pltpu.VMEM((CAP,), jnp.float32),
        pltpu.VMEM((CAP,), jnp.int32),
        pltpu.VMEM((CAP,), jnp.float32),
        pltpu.VMEM((GV,), jnp.float32),
        pltpu.VMEM((GV,), jnp.int32),
        pltpu.VMEM((GV,), jnp.float32),
        pltpu.VMEM((L,), jnp.float32),
        pltpu.SemaphoreType.DMA,
    ],
)
def _sc_topk(s_hbm, t_hbm, out_hbm, win, cval, cidx, pbuf, gval, gidx,
             sdst, res, sem):
    _sc_body(s_hbm, t_hbm, out_hbm, win, cval, cidx, pbuf, gval, gidx,
             sdst, res, sem)


def _cos_body(x_ref, o_ref):
    x = x_ref[...]
    dot = x[:, 0:1]
    ss = x[:, 1:2]
    tt = x[:, 2:3]
    nx = jnp.maximum(jnp.sqrt(ss), EPS)
    ny = jnp.maximum(jnp.sqrt(tt), EPS)
    cos = dot / (nx * ny)
    o_ref[...] = (jnp.sum(1.0 - cos) / B).reshape(1, 1)


def kernel(s_logits, t_logits):
    stats = _sc_topk(s_logits.reshape(-1), t_logits.reshape(-1))
    out = pl.pallas_call(
        _cos_body,
        out_shape=jax.ShapeDtypeStruct((1, 1), jnp.float32),
    )(stats.reshape(B, L))
    return out[0, 0]


# speedup vs baseline: 10.4756x; 1.4656x over previous
import functools

import jax
import jax.numpy as jnp
from jax import lax
from jax.experimental import pallas as pl
from jax.experimental.pallas import tpu as pltpu
from jax.experimental.pallas import tpu_sc as plsc

B = 128
V = 100000
K = 100
EPS = 1e-8

NC = 2
NS = 16
NW = NC * NS
RPW = B // NW
WIN = 20000
NWIN = V // WIN
GRP = 160
NGRP = WIN // GRP
NVR = GRP // 16
CAP = 1024
GPAD = 128
L = 16


def _popc(m):
    return plsc.all_reduce_population_count(m)[0]


def _sc_body(s_hbm, t_hbm, out_hbm, win, cval, cidx, pbuf, gval, gidx,
             sdst, res, sem):
    wid = lax.axis_index("s") * NC + lax.axis_index("c")
    lane = lax.iota(jnp.int32, 16)

    def select_thr(n):
        def cp(i, c):
            pbuf[pl.ds(i * 16, 16)] = cval[pl.ds(i * 16, 16)]
            return c

        lax.fori_loop(0, CAP // 16, cp, 0)

        def w_cond(c):
            return jnp.logical_not(c[0])

        def w_body(c):
            _, an, r, thr, m = c
            v0 = pbuf[pl.ds(0, 16)]
            pivot = v0[0]
            nv = (an + 15) // 16

            def cnt(i, cc):
                cgt, ceq = cc
                x = pbuf[pl.ds(i * 16, 16)]
                valid = (lane + i * 16) < an
                gt = jnp.logical_and(valid, x > pivot)
                eq = jnp.logical_and(valid, x == pivot)
                return cgt + _popc(gt), ceq + _popc(eq)

            cgt, ceq = lax.fori_loop(0, nv, cnt,
                                     (jnp.int32(0), jnp.int32(0)))
            done = jnp.logical_and(cgt <= r, cgt + ceq > r)

            def finish(an, r, thr, m):
                return (jnp.bool_(True), an, r, pivot, r - cgt + 1)

            def recurse(an, r, thr, m):
                up = cgt > r

                def cmp(i, kp):
                    x = pbuf[pl.ds(i * 16, 16)]
                    valid = (lane + i * 16) < an
                    gt = jnp.logical_and(valid, x > pivot)
                    lt = jnp.logical_and(valid, x < pivot)
                    keep = jnp.logical_or(
                        jnp.logical_and(gt, up),
                        jnp.logical_and(lt, jnp.logical_not(up)))
                    plsc.store_compressed(pbuf.at[pl.ds(kp, 16)], x,
                                          mask=keep)
                    return kp + _popc(keep)

                kp = lax.fori_loop(0, nv, cmp, jnp.int32(0))
                r2 = jnp.where(up, r, r - (cgt + ceq))
                return (jnp.bool_(False), kp, r2, thr, m)

            return lax.cond(done, finish, recurse, an, r, thr, m)

        carry = (jnp.bool_(False), n, jnp.int32(K - 1), jnp.float32(0.0),
                 jnp.int32(0))
        _, _, _, thr, m = lax.while_loop(w_cond, w_body, carry)
        return thr, m

    def compact(n, thr, m, dval, didx):
        nv = (n + 15) // 16

        def body(i, cc):
            kp, eqt = cc
            x = cval[pl.ds(i * 16, 16)]
            ix = cidx[pl.ds(i * 16, 16)]
            valid = (lane + i * 16) < n
            gt = jnp.logical_and(valid, x > thr)
            eq = jnp.logical_and(valid, x == thr)
            eqc = plsc.cumsum(eq.astype(jnp.int32)) + eqt
            take = jnp.logical_and(eq, eqc <= m)
            keep = jnp.logical_or(gt, take)
            plsc.store_compressed(dval.at[pl.ds(kp, 16)], x, mask=keep)
            plsc.store_compressed(didx.at[pl.ds(kp, 16)], ix, mask=keep)
            return kp + _popc(keep), eqt + _popc(eq)

        lax.fori_loop(0, nv, body, (jnp.int32(0), jnp.int32(0)))

    def row_body(j, _):
        r = wid * RPW + j
        sbase = r * V

        def win_body(w, carry):
            ptr, thr = carry
            pltpu.sync_copy(t_hbm.at[pl.ds(sbase + w * WIN, WIN)], win)

            def grp(g, carry):
                ptr, thr = carry
                off = g * GRP
                xs = [win[pl.ds(off + q * 16, 16)] for q in range(NVR)]
                anyv = xs[0] > thr
                for q in range(1, NVR):
                    anyv = jnp.logical_or(anyv, xs[q] > thr)
                hit = _popc(anyv) > 0

                def rebuild(ptr0, thr0):
                    t2, m2 = select_thr(ptr0)
                    compact(ptr0, t2, m2, cval, cidx)
                    return jnp.int32(K), t2

                def keep_pt(ptr0, thr0):
                    return ptr0, thr0

                full = jnp.logical_and(hit, ptr + GRP > CAP)
                ptr, thr = lax.cond(full, rebuild, keep_pt, ptr, thr)

                def do_append(p):
                    pp = p
                    for q in range(NVR):
                        x = xs[q]
                        mm = x > thr
                        plsc.store_compressed(cval.at[pl.ds(pp, 16)], x,
                                              mask=mm)
                        colv = sbase + w * WIN + off + q * 16 + lane
                        plsc.store_compressed(cidx.at[pl.ds(pp, 16)], colv,
                                              mask=mm)
                        pp = pp + _popc(mm)
                    return pp

                ptr = lax.cond(hit, do_append, lambda p: p, ptr)
                return ptr, thr

            return lax.fori_loop(0, NGRP, grp, (ptr, thr))

        ptr, thr = lax.fori_loop(0, NWIN, win_body,
                                 (jnp.int32(0), jnp.float32(-jnp.inf)))

        for q in range(GPAD // 16):
            gval[pl.ds(q * 16, 16)] = jnp.zeros((16,), jnp.float32)
            gidx[pl.ds(q * 16, 16)] = jnp.zeros((16,), jnp.int32) + sbase

        t2, m2 = select_thr(ptr)
        compact(ptr, t2, m2, gval, gidx)

        pltpu.async_copy(s_hbm.at[gidx], sdst, sem).wait()

        def acc(q, cc):
            d, ss, tt = cc
            tv = gval[pl.ds(q * 16, 16)]
            sv = sdst[pl.ds(q * 16, 16)]
            valid = (lane + q * 16) < K
            sv = jnp.where(valid, sv, 0.0)
            return d + sv * tv, ss + sv * sv, tt + tv * tv

        z = jnp.zeros((16,), jnp.float32)
        dv, ssv, ttv = lax.fori_loop(0, 7, acc, (z, z, z))
        dot = jnp.sum(dv)
        ssum = jnp.sum(ssv)
        tsum = jnp.sum(ttv)
        res[...] = jnp.where(
            lane == 0, dot,
            jnp.where(lane == 1, ssum, jnp.where(lane == 2, tsum, 0.0)))
        pltpu.sync_copy(res, out_hbm.at[pl.ds(r * L, L)])
        return 0

    lax.fori_loop(0, RPW, row_body, 0)


@functools.partial(
    pl.kernel,
    out_type=jax.ShapeDtypeStruct((B * L,), jnp.float32),
    mesh=plsc.VectorSubcoreMesh(core_axis_name="c", subcore_axis_name="s"),
    compiler_params=pltpu.CompilerParams(needs_layout_passes=False),
    scratch_types=[
        pltpu.VMEM((WIN,), jnp.float32),
        pltpu.VMEM((CAP + 16,), jnp.float32),
        pltpu.VMEM((CAP + 16,), jnp.int32),
        pltpu.VMEM((CAP + 16,), jnp.float32),
        pltpu.VMEM((GPAD,), jnp.float32),
        pltpu.VMEM((GPAD,), jnp.int32),
        pltpu.VMEM((GPAD,), jnp.float32),
        pltpu.VMEM((L,), jnp.float32),
        pltpu.SemaphoreType.DMA,
    ],
)
def _sc_topk(s_hbm, t_hbm, out_hbm, win, cval, cidx, pbuf, gval, gidx,
             sdst, res, sem):
    _sc_body(s_hbm, t_hbm, out_hbm, win, cval, cidx, pbuf, gval, gidx,
             sdst, res, sem)


def _cos_body(x_ref, o_ref):
    x = x_ref[...]
    dot = x[:, 0:1]
    ss = x[:, 1:2]
    tt = x[:, 2:3]
    nx = jnp.maximum(jnp.sqrt(ss), EPS)
    ny = jnp.maximum(jnp.sqrt(tt), EPS)
    cos = dot / (nx * ny)
    o_ref[...] = (jnp.sum(1.0 - cos) / B).reshape(1, 1)


def kernel(s_logits, t_logits):
    stats = _sc_topk(s_logits.reshape(-1), t_logits.reshape(-1))
    out = pl.pallas_call(
        _cos_body,
        out_shape=jax.ShapeDtypeStruct((1, 1), jnp.float32),
    )(stats.reshape(B, L))
    return out[0, 0]


# speedup vs baseline: 11.4242x; 1.0906x over previous
import functools

import jax
import jax.numpy as jnp
from jax import lax
from jax.experimental import pallas as pl
from jax.experimental.pallas import tpu as pltpu
from jax.experimental.pallas import tpu_sc as plsc

B = 128
V = 100000
K = 100
EPS = 1e-8

NC = 2
NS = 16
NW = NC * NS
RPW = B // NW
WIN = 20000
NWIN = V // WIN
GRP = 400
NGRP = WIN // GRP
NVR = GRP // 16
CAP = 2048
GPAD = 128
L = 16


def _popc(m):
    return plsc.all_reduce_population_count(m)[0]


def _sc_body(s_hbm, t_hbm, out_hbm, win, cval, cidx, pbuf, gval, gidx,
             sdst, res, sem):
    wid = lax.axis_index("s") * NC + lax.axis_index("c")
    lane = lax.iota(jnp.int32, 16)

    def select_thr(n):
        def cp(i, c):
            pbuf[pl.ds(i * 16, 16)] = cval[pl.ds(i * 16, 16)]
            return c

        lax.fori_loop(0, CAP // 16, cp, 0)

        def w_cond(c):
            return jnp.logical_not(c[0])

        def w_body(c):
            _, an, r, thr, m = c
            v0 = pbuf[pl.ds(0, 16)]
            pivot = v0[0]
            nv = (an + 15) // 16

            def cnt(i, cc):
                cgt, ceq = cc
                x = pbuf[pl.ds(i * 16, 16)]
                valid = (lane + i * 16) < an
                gt = jnp.logical_and(valid, x > pivot)
                eq = jnp.logical_and(valid, x == pivot)
                return cgt + _popc(gt), ceq + _popc(eq)

            cgt, ceq = lax.fori_loop(0, nv, cnt,
                                     (jnp.int32(0), jnp.int32(0)))
            done = jnp.logical_and(cgt <= r, cgt + ceq > r)

            def finish(an, r, thr, m):
                return (jnp.bool_(True), an, r, pivot, r - cgt + 1)

            def recurse(an, r, thr, m):
                up = cgt > r

                def cmp(i, kp):
                    x = pbuf[pl.ds(i * 16, 16)]
                    valid = (lane + i * 16) < an
                    gt = jnp.logical_and(valid, x > pivot)
                    lt = jnp.logical_and(valid, x < pivot)
                    keep = jnp.logical_or(
                        jnp.logical_and(gt, up),
                        jnp.logical_and(lt, jnp.logical_not(up)))
                    plsc.store_compressed(pbuf.at[pl.ds(kp, 16)], x,
                                          mask=keep)
                    return kp + _popc(keep)

                kp = lax.fori_loop(0, nv, cmp, jnp.int32(0))
                r2 = jnp.where(up, r, r - (cgt + ceq))
                return (jnp.bool_(False), kp, r2, thr, m)

            return lax.cond(done, finish, recurse, an, r, thr, m)

        carry = (jnp.bool_(False), n, jnp.int32(K - 1), jnp.float32(0.0),
                 jnp.int32(0))
        _, _, _, thr, m = lax.while_loop(w_cond, w_body, carry)
        return thr, m

    def compact(n, thr, m, dval, didx):
        nv = (n + 15) // 16

        def body(i, cc):
            kp, eqt = cc
            x = cval[pl.ds(i * 16, 16)]
            ix = cidx[pl.ds(i * 16, 16)]
            valid = (lane + i * 16) < n
            gt = jnp.logical_and(valid, x > thr)
            eq = jnp.logical_and(valid, x == thr)
            eqc = plsc.cumsum(eq.astype(jnp.int32)) + eqt
            take = jnp.logical_and(eq, eqc <= m)
            keep = jnp.logical_or(gt, take)
            plsc.store_compressed(dval.at[pl.ds(kp, 16)], x, mask=keep)
            plsc.store_compressed(didx.at[pl.ds(kp, 16)], ix, mask=keep)
            return kp + _popc(keep), eqt + _popc(eq)

        lax.fori_loop(0, nv, body, (jnp.int32(0), jnp.int32(0)))

    def row_body(j, _):
        r = wid * RPW + j
        sbase = r * V

        def win_body(w, carry):
            ptr, thr = carry
            pltpu.sync_copy(t_hbm.at[pl.ds(sbase + w * WIN, WIN)], win)

            def grp(g, carry):
                ptr, thr = carry
                off = g * GRP
                xs = [win[pl.ds(off + q * 16, 16)] for q in range(NVR)]
                ms = [x > thr for x in xs]
                while len(ms) > 1:
                    ms = [jnp.logical_or(a, b) for a, b in zip(ms[::2], ms[1::2])] + (
                        [ms[-1]] if len(ms) % 2 else [])
                hit = _popc(ms[0]) > 0

                def hit_path(ptr0, thr0):
                    def rebuild(p0, t0):
                        t2, m2 = select_thr(p0)
                        compact(p0, t2, m2, cval, cidx)
                        return jnp.int32(K), t2

                    def keep_pt(p0, t0):
                        return p0, t0

                    p1, t1 = lax.cond(ptr0 + GRP > CAP, rebuild, keep_pt,
                                      ptr0, thr0)
                    pp = p1
                    for q in range(NVR):
                        x = xs[q]
                        mm = x > t1
                        plsc.store_compressed(cval.at[pl.ds(pp, 16)], x,
                                              mask=mm)
                        colv = sbase + w * WIN + off + q * 16 + lane
                        plsc.store_compressed(cidx.at[pl.ds(pp, 16)], colv,
                                              mask=mm)
                        pp = pp + _popc(mm)
                    return pp, t1

                def miss_path(ptr0, thr0):
                    return ptr0, thr0

                ptr, thr = lax.cond(hit, hit_path, miss_path, ptr, thr)
                return ptr, thr

            return lax.fori_loop(0, NGRP, grp, (ptr, thr))

        ptr, thr = lax.fori_loop(0, NWIN, win_body,
                                 (jnp.int32(0), jnp.float32(-jnp.inf)))

        for q in range(GPAD // 16):
            gval[pl.ds(q * 16, 16)] = jnp.zeros((16,), jnp.float32)
            gidx[pl.ds(q * 16, 16)] = jnp.zeros((16,), jnp.int32) + sbase

        t2, m2 = select_thr(ptr)
        compact(ptr, t2, m2, gval, gidx)

        pltpu.async_copy(s_hbm.at[gidx], sdst, sem).wait()

        def acc(q, cc):
            d, ss, tt = cc
            tv = gval[pl.ds(q * 16, 16)]
            sv = sdst[pl.ds(q * 16, 16)]
            valid = (lane + q * 16) < K
            sv = jnp.where(valid, sv, 0.0)
            return d + sv * tv, ss + sv * sv, tt + tv * tv

        z = jnp.zeros((16,), jnp.float32)
        dv, ssv, ttv = lax.fori_loop(0, 7, acc, (z, z, z))
        dot = jnp.sum(dv)
        ssum = jnp.sum(ssv)
        tsum = jnp.sum(ttv)
        res[...] = jnp.where(
            lane == 0, dot,
            jnp.where(lane == 1, ssum, jnp.where(lane == 2, tsum, 0.0)))
        pltpu.sync_copy(res, out_hbm.at[pl.ds(r * L, L)])
        return 0

    lax.fori_loop(0, RPW, row_body, 0)


@functools.partial(
    pl.kernel,
    out_type=jax.ShapeDtypeStruct((B * L,), jnp.float32),
    mesh=plsc.VectorSubcoreMesh(core_axis_name="c", subcore_axis_name="s"),
    compiler_params=pltpu.CompilerParams(needs_layout_passes=False),
    scratch_types=[
        pltpu.VMEM((WIN,), jnp.float32),
        pltpu.VMEM((CAP + 16,), jnp.float32),
        pltpu.VMEM((CAP + 16,), jnp.int32),
        pltpu.VMEM((CAP + 16,), jnp.float32),
        pltpu.VMEM((GPAD,), jnp.float32),
        pltpu.VMEM((GPAD,), jnp.int32),
        pltpu.VMEM((GPAD,), jnp.float32),
        pltpu.VMEM((L,), jnp.float32),
        pltpu.SemaphoreType.DMA,
    ],
)
def _sc_topk(s_hbm, t_hbm, out_hbm, win, cval, cidx, pbuf, gval, gidx,
             sdst, res, sem):
    _sc_body(s_hbm, t_hbm, out_hbm, win, cval, cidx, pbuf, gval, gidx,
             sdst, res, sem)


def _cos_body(x_ref, o_ref):
    x = x_ref[...]
    dot = x[:, 0:1]
    ss = x[:, 1:2]
    tt = x[:, 2:3]
    nx = jnp.maximum(jnp.sqrt(ss), EPS)
    ny = jnp.maximum(jnp.sqrt(tt), EPS)
    cos = dot / (nx * ny)
    o_ref[...] = (jnp.sum(1.0 - cos) / B).reshape(1, 1)


def kernel(s_logits, t_logits):
    stats = _sc_topk(s_logits.reshape(-1), t_logits.reshape(-1))
    out = pl.pallas_call(
        _cos_body,
        out_shape=jax.ShapeDtypeStruct((1, 1), jnp.float32),
    )(stats.reshape(B, L))
    return out[0, 0]
